# baseline (speedup 1.0000x reference)
import functools

import jax
import jax.numpy as jnp
import numpy as np
from jax import lax
from jax.experimental import pallas as pl
from jax.experimental.pallas import tpu as pltpu
from jax.experimental.pallas import tpu_sc as plsc

ALPHA = 4
D = 64
SEQ_LEN = 65536
NW = 16
LANES = 16
NCHAINS = NW * LANES
CHAIN_LEN = SEQ_LEN // NCHAINS
W_LEN = LANES * CHAIN_LEN
REC = 96

TB_SGN = 0
TB_CSE = 64
TB_DW = 80
TB_BIAS = 144
TB_CHM = 208
TB_W1T = 464
TB_B1 = 4560
TB_W2T = 4624
TB_B2 = 8720
TB_E = 8784
TB_ZB = 12880
TB_LEN = 12944


def _merge16(rec_v, rbase):
    zeros = jnp.zeros((16,), jnp.float32)
    lane0 = lax.iota(jnp.int32, 16)
    idx0 = [lane0 >> 2,
            lane0 & 3]
    idx0 += [4 + (lane0 & 3) * 4 + c for c in range(ALPHA)]
    idx0 += [jnp.full((16,), c, jnp.int32) for c in range(ALPHA)]
    idx0 = [ix + rbase for ix in idx0]

    def mbody(_, carry):
        (off, w1e, w2, w30, w31, w32, w33) = carry[:7]
        ixs = list(carry[7:])
        w3 = [w30, w31, w32, w33]
        r1e = plsc.load_gather(rec_v, [ixs[0]])
        r1b = plsc.load_gather(rec_v, [ixs[1]])
        r2p = [plsc.load_gather(rec_v, [ixs[2 + c]]) for c in range(ALPHA)]
        r1s = [plsc.load_gather(rec_v, [ixs[6 + c]]) for c in range(ALPHA)]
        r2 = rec_v[pl.ds(off + 4, 16)]
        w3n = []
        for c in range(ALPHA):
            r3c = rec_v[pl.ds(off + 20 + c * 16, 16)]
            w3n.append(w3[c] + r3c + w2 * r1s[c] + w1e * r2p[c])
        w2n = w2 + r2 + w1e * r1b
        w1n = w1e + r1e
        return tuple([off + REC, w1n, w2n] + w3n + [ix + REC for ix in ixs])

    carry = lax.fori_loop(
        0, NW, mbody,
        tuple([rbase, zeros, zeros, zeros, zeros, zeros, zeros] + idx0))
    return carry[1], carry[2], [carry[3], carry[4], carry[5], carry[6]]


def _sc_all(seq_hbm, tab_hbm, out_hbm,
            seq_v, c3_v, rec_v, pbuf_v, shared_v, gbuf_v, tab_v,
            scr_v, obuf_v):
    wid = lax.axis_index("s")
    pltpu.sync_copy(seq_hbm.at[pl.ds(wid * W_LEN, W_LEN)], seq_v)

    lane = lax.iota(jnp.int32, 16)
    zeros = jnp.zeros((16,), jnp.float32)
    for j in range(LANES * D // 16):
        c3_v[pl.ds(j * 16, 16)] = zeros

    def body(_, carry):
        gidx = carry[0]
        C1 = list(carry[1:5])
        C2 = list(carry[5:21])
        ln = lax.iota(jnp.int32, 16)
        xv = plsc.load_gather(seq_v, [gidx])
        base = (xv << 4) + ln
        m = [(xv == b).astype(jnp.float32) for b in range(ALPHA)]
        for ab in range(16):
            plsc.addupdate_scatter(c3_v, [base + (ab * 64)], C2[ab])
        for a in range(ALPHA):
            for b in range(ALPHA):
                C2[a * 4 + b] = C2[a * 4 + b] + C1[a] * m[b]
        for a in range(ALPHA):
            C1[a] = C1[a] + m[a]
        return tuple([gidx + 1] + C1 + C2)

    gidx0 = lane * CHAIN_LEN
    carry = lax.fori_loop(0, CHAIN_LEN, body, tuple([gidx0] + [zeros] * 20))
    C1 = carry[1:5]
    C2 = carry[5:21]

    riota = lane * REC
    for a in range(ALPHA):
        plsc.store_scatter(rec_v, [riota + a], C1[a])
    for ab in range(16):
        plsc.store_scatter(rec_v, [riota + (4 + ab)], C2[ab])
    for ab in range(16):
        for c in range(ALPHA):
            v = c3_v[pl.ds(ab * 64 + c * 16, 16)]
            plsc.store_scatter(rec_v, [riota + (20 + c * 16 + ab)], v)

    w1e, w2, w3 = _merge16(rec_v, 0)
    for j in range(6):
        pbuf_v[pl.ds(j * 16, 16)] = zeros
    b0 = (lane & 3) == 0
    park = 84 + lane - 1 - (lane >> 2)
    plsc.store_scatter(pbuf_v, [jnp.where(b0, lane >> 2, park)], w1e)
    pbuf_v[pl.ds(4, 16)] = w2
    for c in range(ALPHA):
        pbuf_v[pl.ds(20 + c * 16, 16)] = w3[c]
    pltpu.sync_copy(pbuf_v, shared_v.at[pl.ds(wid * REC, REC)])
    plsc.subcore_barrier()

    @pl.when(wid == 0)
    def _():
        pltpu.sync_copy(shared_v, gbuf_v)
        pltpu.sync_copy(gbuf_v, out_hbm)
        pltpu.sync_copy(tab_hbm, tab_v)
        g1e, _g2, g3 = _merge16(gbuf_v, 0)

        ln = lax.iota(jnp.int32, 16)
        zz = jnp.zeros((16,), jnp.float32)

        def reduce_splat(vec):
            scr_v[pl.ds(0, 16)] = vec
            acc = vec
            for sh in (8, 4, 2, 1):
                other = plsc.load_gather(scr_v, [(ln + sh) & 15])
                acc = acc + other
                scr_v[pl.ds(0, 16)] = acc
            return plsc.load_gather(scr_v, [jnp.zeros((16,), jnp.int32)])

        scal = reduce_splat(g1e * tab_v[pl.ds(TB_CSE, 16)])

        scr_v[pl.ds(0, 16)] = g1e
        c1s = [plsc.load_gather(scr_v, [jnp.full((16,), a * 4, jnp.int32)])
               for a in range(ALPHA)]

        def dense(xs, woff, boff, relu):
            for j in range(4):
                scr_v[pl.ds(j * 16, 16)] = xs[j]
            acc = [tab_v[pl.ds(boff + j * 16, 16)] for j in range(4)]
            for k in range(D):
                xk = plsc.load_gather(scr_v, [jnp.full((16,), k, jnp.int32)])
                for j in range(4):
                    acc[j] = acc[j] + xk * tab_v[pl.ds(woff + k * 64 + j * 16, 16)]
            if relu:
                acc = [jnp.maximum(a, 0.0) for a in acc]
            return acc

        coef = [g3[c] * tab_v[pl.ds(TB_SGN + c * 16, 16)] for c in range(ALPHA)]
        base = dense(coef, TB_E, TB_ZB, False)

        enh = []
        for j in range(4):
            dwj = tab_v[pl.ds(TB_DW + j * 16, 16)]
            bij = tab_v[pl.ds(TB_BIAS + j * 16, 16)]
            mj = zz
            for a in range(ALPHA):
                mj = mj + c1s[a] * tab_v[pl.ds(TB_CHM + a * 64 + j * 16, 16)]
            enh.append((base[j] * dwj + bij) * scal + mj)

        hid = dense(enh, TB_W1T, TB_B1, True)
        out = dense(hid, TB_W2T, TB_B2, False)
        for j in range(4):
            obuf_v[pl.ds(j * 16, 16)] = out[j]


def _tc_merge_kernel(part_ref, idx_ref, sgn_ref, cs_ref, chm_ref, dw_ref, bias_ref,
                     w1t_ref, b1_ref, w2t_ref, b2_ref, out_ref):
    Prt = part_ref[:]
    C1 = Prt[:, 0:4]
    c2_blk = Prt[:, 4:20]
    c3_blk = Prt[:, 20:84]
    mi = lax.broadcasted_iota(jnp.int32, (D, D), 0)
    ki = lax.broadcasted_iota(jnp.int32, (D, D), 1)
    M = jnp.where(ki == (((mi >> 2) & 3) * 16 + (mi & 3) * 4 + (mi >> 4)), 1.0, 0.0)
    C3 = jnp.dot(c3_blk, M, preferred_element_type=jnp.float32)

    il_r = lax.broadcasted_iota(jnp.int32, (NW, NW), 0)
    il_c = lax.broadcasted_iota(jnp.int32, (NW, NW), 1)
    L = (il_c < il_r).astype(jnp.float32)

    def bcast_cols(x, reps):
        return jnp.concatenate(
            [jnp.broadcast_to(x[:, j:j + 1], (x.shape[0], reps))
             for j in range(x.shape[1])], axis=1)

    pre1 = jnp.dot(L, C1, preferred_element_type=jnp.float32)
    q2inc = c2_blk + bcast_cols(pre1, 4) * jnp.concatenate([C1] * 4, axis=1)
    pre2 = jnp.dot(L, q2inc, preferred_element_type=jnp.float32)
    c3contrib = (C3
                 + bcast_cols(pre2, 4) * jnp.concatenate([C1] * 16, axis=1)
                 + bcast_cols(pre1, 16) * jnp.concatenate([c2_blk] * 4, axis=1))
    c3row = jnp.sum(c3contrib, axis=0, keepdims=True)
    c1row = jnp.sum(C1, axis=0, keepdims=True)

    ed = lax.broadcasted_iota(jnp.int32, (D, D), 1)
    E = jnp.where(ed == idx_ref[:], sgn_ref[:], 0.0)
    baseline = jnp.dot(c3row, E, preferred_element_type=jnp.float32)
    inv_n = 1.0 / SEQ_LEN
    scaling = jnp.sum(c1row * cs_ref[:], axis=1, keepdims=True) * inv_n
    mods = jnp.dot(c1row, chm_ref[:], preferred_element_type=jnp.float32) * inv_n
    enhanced = (baseline * dw_ref[:] + bias_ref[:]) * scaling + mods
    hidden = jnp.maximum(
        jnp.dot(enhanced, w1t_ref[:], preferred_element_type=jnp.float32) + b1_ref[:], 0.0)
    out_ref[:] = jnp.dot(hidden, w2t_ref[:], preferred_element_type=jnp.float32) + b2_ref[:]


@functools.cache
def _sc_all_call():
    return pl.kernel(
        _sc_all,
        out_type=jax.ShapeDtypeStruct((NW * REC,), jnp.float32),
        mesh=plsc.VectorSubcoreMesh(
            core_axis_name="c", subcore_axis_name="s", num_cores=1),
        compiler_params=pltpu.CompilerParams(needs_layout_passes=False),
        scratch_types=[
            pltpu.VMEM((W_LEN,), jnp.int32),
            pltpu.VMEM((LANES * D,), jnp.float32),
            pltpu.VMEM((LANES * REC,), jnp.float32),
            pltpu.VMEM((REC,), jnp.float32),
            pltpu.VMEM_SHARED((NW * REC,), jnp.float32),
            pltpu.VMEM((NW * REC,), jnp.float32),
            pltpu.VMEM((TB_LEN,), jnp.float32),
            pltpu.VMEM((D,), jnp.float32),
            pltpu.VMEM((D,), jnp.float32),
        ],
    )


def kernel(sequence, h_hash, s_signs, char_scales, dimension_weights, sketch_bias,
           char_hash_modifiers, W1, b1, W2, b2):
    hsum = (h_hash[2][:, None, None] + h_hash[0][None, :, None]
            + h_hash[1][None, None, :]) % D
    idx64 = jnp.reshape(hsum, (D,)).astype(jnp.int32)
    sgn64 = jnp.reshape(
        s_signs[2][:, None, None] * s_signs[0][None, :, None]
        * s_signs[1][None, None, :], (D,))
    inv_n = 1.0 / SEQ_LEN
    cs_e = jnp.tile(char_scales[:, None], (1, 4)).reshape(16) * (0.25 * inv_n)
    tab = jnp.concatenate([
        sgn64,
        cs_e,
        dimension_weights,
        sketch_bias,
        (char_hash_modifiers * inv_n).reshape(-1),
        W1.T.reshape(-1),
        b1,
        W2.T.reshape(-1),
        b2,
        (idx64[:, None] == jnp.arange(D, dtype=jnp.int32)[None, :])
        .astype(jnp.float32).reshape(-1),
        jnp.zeros((D,), jnp.float32),
    ])
    partials = _sc_all_call()(sequence, tab).reshape(NW, REC)

    idx64k = jnp.reshape(
        (h_hash[0][:, None, None] + h_hash[1][None, :, None]
         + h_hash[2][None, None, :]) % D, (D, 1)).astype(jnp.int32)
    sgn64k = jnp.reshape(
        s_signs[0][:, None, None] * s_signs[1][None, :, None]
        * s_signs[2][None, None, :], (D, 1))
    out = pl.pallas_call(
        _tc_merge_kernel,
        out_shape=jax.ShapeDtypeStruct((1, D), jnp.float32),
    )(partials, idx64k, sgn64k,
      char_scales.reshape(1, ALPHA), char_hash_modifiers,
      dimension_weights.reshape(1, D), sketch_bias.reshape(1, D),
      W1.T, b1.reshape(1, D), W2.T, b2.reshape(1, D))
    return out.reshape(D)


# speedup vs baseline: 1.0889x; 1.0889x over previous
import functools

import jax
import jax.numpy as jnp
import numpy as np
from jax import lax
from jax.experimental import pallas as pl
from jax.experimental.pallas import tpu as pltpu
from jax.experimental.pallas import tpu_sc as plsc

ALPHA = 4
D = 64
SEQ_LEN = 65536
NW = 16
LANES = 16
NCHAINS = NW * LANES
CHAIN_LEN = SEQ_LEN // NCHAINS
W_LEN = LANES * CHAIN_LEN
REC = 96

TB_SGN = 0
TB_CSE = 64
TB_DW = 80
TB_BIAS = 144
TB_CHM = 208
TB_W1T = 464
TB_B1 = 4560
TB_W2T = 4624
TB_B2 = 8720
TB_E = 8784
TB_ZB = 12880
TB_LEN = 12944


def _merge16(rec_v, rbase):
    zeros = jnp.zeros((16,), jnp.float32)
    lane0 = lax.iota(jnp.int32, 16)
    idx0 = [lane0 >> 2,
            lane0 & 3]
    idx0 += [4 + (lane0 & 3) * 4 + c for c in range(ALPHA)]
    idx0 += [jnp.full((16,), c, jnp.int32) for c in range(ALPHA)]
    idx0 = [ix + rbase for ix in idx0]

    def mbody(_, carry):
        (off, w1e, w2, w30, w31, w32, w33) = carry[:7]
        ixs = list(carry[7:])
        w3 = [w30, w31, w32, w33]
        r1e = plsc.load_gather(rec_v, [ixs[0]])
        r1b = plsc.load_gather(rec_v, [ixs[1]])
        r2p = [plsc.load_gather(rec_v, [ixs[2 + c]]) for c in range(ALPHA)]
        r1s = [plsc.load_gather(rec_v, [ixs[6 + c]]) for c in range(ALPHA)]
        r2 = rec_v[pl.ds(off + 4, 16)]
        w3n = []
        for c in range(ALPHA):
            r3c = rec_v[pl.ds(off + 20 + c * 16, 16)]
            w3n.append(w3[c] + r3c + w2 * r1s[c] + w1e * r2p[c])
        w2n = w2 + r2 + w1e * r1b
        w1n = w1e + r1e
        return tuple([off + REC, w1n, w2n] + w3n + [ix + REC for ix in ixs])

    carry = lax.fori_loop(
        0, NW, mbody,
        tuple([rbase, zeros, zeros, zeros, zeros, zeros, zeros] + idx0))
    return carry[1], carry[2], [carry[3], carry[4], carry[5], carry[6]]


def _sc_all(seq_hbm, tab_hbm, out_hbm,
            seq_v, c3_v, rec_v, pbuf_v, shared_v, gbuf_v, tab_v,
            scr_v, obuf_v):
    wid = lax.axis_index("s")
    pltpu.sync_copy(seq_hbm.at[pl.ds(wid * W_LEN, W_LEN)], seq_v)

    lane = lax.iota(jnp.int32, 16)
    zeros = jnp.zeros((16,), jnp.float32)
    for j in range(LANES * D // 16):
        c3_v[pl.ds(j * 16, 16)] = zeros

    def body(_, carry):
        gidx = carry[0]
        C1 = list(carry[1:5])
        C2 = list(carry[5:21])
        ln = lax.iota(jnp.int32, 16)
        xv = plsc.load_gather(seq_v, [gidx])
        base = (xv << 4) + ln
        m = [(xv == b).astype(jnp.float32) for b in range(ALPHA)]
        for ab in range(16):
            plsc.addupdate_scatter(c3_v, [base + (ab * 64)], C2[ab])
        for a in range(ALPHA):
            for b in range(ALPHA):
                C2[a * 4 + b] = C2[a * 4 + b] + C1[a] * m[b]
        for a in range(ALPHA):
            C1[a] = C1[a] + m[a]
        return tuple([gidx + 1] + C1 + C2)

    gidx0 = lane * CHAIN_LEN
    carry = lax.fori_loop(0, CHAIN_LEN, body, tuple([gidx0] + [zeros] * 20))
    C1 = carry[1:5]
    C2 = carry[5:21]

    riota = lane * REC
    for a in range(ALPHA):
        plsc.store_scatter(rec_v, [riota + a], C1[a])
    for ab in range(16):
        plsc.store_scatter(rec_v, [riota + (4 + ab)], C2[ab])
    for ab in range(16):
        for c in range(ALPHA):
            v = c3_v[pl.ds(ab * 64 + c * 16, 16)]
            plsc.store_scatter(rec_v, [riota + (20 + c * 16 + ab)], v)

    w1e, w2, w3 = _merge16(rec_v, 0)
    for j in range(6):
        pbuf_v[pl.ds(j * 16, 16)] = zeros
    b0 = (lane & 3) == 0
    park = 84 + lane - 1 - (lane >> 2)
    plsc.store_scatter(pbuf_v, [jnp.where(b0, lane >> 2, park)], w1e)
    pbuf_v[pl.ds(4, 16)] = w2
    for c in range(ALPHA):
        pbuf_v[pl.ds(20 + c * 16, 16)] = w3[c]
    pltpu.sync_copy(pbuf_v, shared_v.at[pl.ds(wid * REC, REC)])
    plsc.subcore_barrier()

    @pl.when(wid == 0)
    def _():
        pltpu.sync_copy(shared_v, gbuf_v)
        pltpu.sync_copy(tab_hbm, tab_v)
        g1e, _g2, g3 = _merge16(gbuf_v, 0)

        zz = jnp.zeros((16,), jnp.float32)

        def splat(vec, l):
            return jnp.full((16,), vec[l], jnp.float32)

        sv = g1e * tab_v[pl.ds(TB_CSE, 16)]
        tot = sv[0]
        for l in range(1, 16):
            tot = tot + sv[l]
        scal = jnp.full((16,), tot, jnp.float32)

        c1s = [splat(g1e, a * 4) for a in range(ALPHA)]

        def dense(xs, woff, boff, relu):
            acc = [tab_v[pl.ds(boff + j * 16, 16)] for j in range(4)]
            for k in range(D):
                xk = splat(xs[k >> 4], k & 15)
                for j in range(4):
                    acc[j] = acc[j] + xk * tab_v[pl.ds(woff + k * 64 + j * 16, 16)]
            if relu:
                acc = [jnp.maximum(a, 0.0) for a in acc]
            return acc

        coef = [g3[c] * tab_v[pl.ds(TB_SGN + c * 16, 16)] for c in range(ALPHA)]
        base = dense(coef, TB_E, TB_ZB, False)

        enh = []
        for j in range(4):
            dwj = tab_v[pl.ds(TB_DW + j * 16, 16)]
            bij = tab_v[pl.ds(TB_BIAS + j * 16, 16)]
            mj = zz
            for a in range(ALPHA):
                mj = mj + c1s[a] * tab_v[pl.ds(TB_CHM + a * 64 + j * 16, 16)]
            enh.append((base[j] * dwj + bij) * scal + mj)

        hid = dense(enh, TB_W1T, TB_B1, True)
        out = dense(hid, TB_W2T, TB_B2, False)
        for j in range(4):
            obuf_v[pl.ds(j * 16, 16)] = out[j]
        pltpu.sync_copy(obuf_v, out_hbm)


@functools.cache
def _sc_all_call():
    return pl.kernel(
        _sc_all,
        out_type=jax.ShapeDtypeStruct((D,), jnp.float32),
        mesh=plsc.VectorSubcoreMesh(
            core_axis_name="c", subcore_axis_name="s", num_cores=1),
        compiler_params=pltpu.CompilerParams(needs_layout_passes=False),
        scratch_types=[
            pltpu.VMEM((W_LEN,), jnp.int32),
            pltpu.VMEM((LANES * D,), jnp.float32),
            pltpu.VMEM((LANES * REC,), jnp.float32),
            pltpu.VMEM((REC,), jnp.float32),
            pltpu.VMEM_SHARED((NW * REC,), jnp.float32),
            pltpu.VMEM((NW * REC,), jnp.float32),
            pltpu.VMEM((TB_LEN,), jnp.float32),
            pltpu.VMEM((D,), jnp.float32),
            pltpu.VMEM((D,), jnp.float32),
        ],
    )


def kernel(sequence, h_hash, s_signs, char_scales, dimension_weights, sketch_bias,
           char_hash_modifiers, W1, b1, W2, b2):
    hsum = (h_hash[2][:, None, None] + h_hash[0][None, :, None]
            + h_hash[1][None, None, :]) % D
    idx64 = jnp.reshape(hsum, (D,)).astype(jnp.int32)
    sgn64 = jnp.reshape(
        s_signs[2][:, None, None] * s_signs[0][None, :, None]
        * s_signs[1][None, None, :], (D,))
    inv_n = 1.0 / SEQ_LEN
    cs_e = jnp.tile(char_scales[:, None], (1, 4)).reshape(16) * (0.25 * inv_n)
    tab = jnp.concatenate([
        sgn64,
        cs_e,
        dimension_weights,
        sketch_bias,
        (char_hash_modifiers * inv_n).reshape(-1),
        W1.T.reshape(-1),
        b1,
        W2.T.reshape(-1),
        b2,
        (idx64[:, None] == jnp.arange(D, dtype=jnp.int32)[None, :])
        .astype(jnp.float32).reshape(-1),
        jnp.zeros((D,), jnp.float32),
    ])
    return _sc_all_call()(sequence, tab)
